# baseline (speedup 1.0000x reference)
import functools

import jax
import jax.numpy as jnp
from jax import lax
from jax.experimental import pallas as pl
from jax.experimental.pallas import tpu as pltpu
from jax.experimental.pallas import tpu_sc as plsc

_LANES = 128


def _make_gather(num_rows: int, num_ids_rows: int, num_ids_cols: int, dim: int):
    info = plsc.get_sparse_core_info()
    nc, ns = info.num_cores, info.num_subcores
    nl = info.num_lanes
    nw = nc * ns
    batch = num_ids_rows * num_ids_cols
    b_per_w = batch // nw
    per_row = num_ids_cols // b_per_w
    mesh = plsc.VectorSubcoreMesh(core_axis_name="c", subcore_axis_name="s")

    chunk = 128
    n_chunks = b_per_w // chunk

    @functools.partial(
        pl.kernel,
        mesh=mesh,
        compiler_params=pltpu.CompilerParams(needs_layout_passes=False),
        out_type=jax.ShapeDtypeStruct((num_ids_rows, dim, num_ids_cols), jnp.float32),
        scratch_types=[
            pltpu.VMEM((b_per_w,), jnp.int32),
            pltpu.VMEM((2, chunk, _LANES), jnp.float32),
            pltpu.VMEM((2, dim, chunk), jnp.float32),
            [pltpu.SemaphoreType.DMA] * 2,
            [pltpu.SemaphoreType.DMA] * 2,
        ],
    )
    def gather_kernel(table_hbm, idx_hbm, out_hbm, idx_v, rows_w, rows_t, gsem, ssem):
        wid = lax.axis_index("s") * nc + lax.axis_index("c")
        r = wid // per_row
        col = (wid % per_row) * b_per_w
        pltpu.sync_copy(idx_hbm.at[r, pl.ds(col, b_per_w)], idx_v)

        lane = lax.iota(jnp.int32, nl)
        diag = [(lane + k) & (nl - 1) for k in range(nl)]

        def start_gather(c):
            pltpu.async_copy(
                table_hbm.at[idx_v.at[pl.ds(c * chunk, chunk)]],
                rows_w.at[c % 2],
                gsem[c % 2],
            )

        def wait_gather(c):
            pltpu.make_async_copy(
                table_hbm.at[idx_v.at[pl.ds(c * chunk, chunk)]],
                rows_w.at[c % 2],
                gsem[c % 2],
            ).wait()

        def start_store(c):
            pltpu.async_copy(
                rows_t.at[c % 2],
                out_hbm.at[r, :, pl.ds(col + c * chunk, chunk)],
                ssem[c % 2],
            )

        def wait_store(c):
            pltpu.make_async_copy(
                rows_t.at[c % 2],
                out_hbm.at[r, :, pl.ds(col + c * chunk, chunk)],
                ssem[c % 2],
            ).wait()

        start_gather(0)
        for c in range(n_chunks):
            b = c % 2
            if c + 1 < n_chunks:
                start_gather(c + 1)
            wait_gather(c)
            if c >= 2:
                wait_store(c - 2)

            def transpose_cblk(cb, carry):
                c_vec = cb * nl + lane
                for db in range(dim // nl):
                    for k in range(nl):
                        d_vec = db * nl + diag[k]
                        v = plsc.load_gather(rows_w.at[b], [c_vec, d_vec])
                        plsc.store_scatter(rows_t.at[b], [d_vec, c_vec], v)
                return carry

            lax.fori_loop(0, chunk // nl, transpose_cblk, 0)
            start_store(c)
        for c in range(n_chunks - 2, n_chunks):
            wait_store(c)

    return gather_kernel


def kernel(embed_id, weight):
    num_rows, dim = weight.shape
    ir, ic = embed_id.shape
    wpad = jnp.pad(weight, ((0, 0), (0, _LANES - dim)))
    out_t = _make_gather(num_rows, ir, ic, dim)(wpad, embed_id.astype(jnp.int32))
    return out_t.transpose(0, 2, 1)


# speedup vs baseline: 1.0707x; 1.0707x over previous
import functools

import jax
import jax.numpy as jnp
from jax import lax
from jax.experimental import pallas as pl
from jax.experimental.pallas import tpu as pltpu
from jax.experimental.pallas import tpu_sc as plsc

_LANES = 128


def _make_gather(num_rows: int, num_ids_rows: int, num_ids_cols: int, dim: int):
    info = plsc.get_sparse_core_info()
    nc, ns = info.num_cores, info.num_subcores
    nw = nc * ns
    batch = num_ids_rows * num_ids_cols
    b_per_w = batch // nw
    per_row = num_ids_cols // b_per_w
    mesh = plsc.VectorSubcoreMesh(core_axis_name="c", subcore_axis_name="s")

    chunk = 256
    n_chunks = b_per_w // chunk

    @functools.partial(
        pl.kernel,
        mesh=mesh,
        out_type=jax.ShapeDtypeStruct((num_ids_rows, num_ids_cols, dim), jnp.float32),
        scratch_types=[
            pltpu.VMEM((b_per_w,), jnp.int32),
            pltpu.VMEM((chunk, _LANES), jnp.float32),
            pltpu.VMEM((chunk, dim), jnp.float32),
            pltpu.SemaphoreType.DMA,
        ],
    )
    def gather_kernel(table_hbm, idx_hbm, out_hbm, idx_v, rows_w, rows_c, sem):
        wid = lax.axis_index("s") * nc + lax.axis_index("c")
        r = wid // per_row
        col = (wid % per_row) * b_per_w
        pltpu.sync_copy(idx_hbm.at[r, pl.ds(col, b_per_w)], idx_v)
        for c in range(n_chunks):
            pltpu.async_copy(
                table_hbm.at[idx_v.at[pl.ds(c * chunk, chunk)]], rows_w, sem
            ).wait()

            def compact_row(i, carry):
                for j in range(dim // 16):
                    rows_c[i, pl.ds(j * 16, 16)] = rows_w[i, pl.ds(j * 16, 16)]
                return carry

            lax.fori_loop(0, chunk, compact_row, 0)
            pltpu.sync_copy(rows_c, out_hbm.at[r, pl.ds(col + c * chunk, chunk)])

    return gather_kernel


def kernel(embed_id, weight):
    num_rows, dim = weight.shape
    ir, ic = embed_id.shape
    wpad = jnp.pad(weight, ((0, 0), (0, _LANES - dim)))
    out = _make_gather(num_rows, ir, ic, dim)(wpad, embed_id.astype(jnp.int32))
    return out


# speedup vs baseline: 1.0984x; 1.0258x over previous
import functools

import jax
import jax.numpy as jnp
from jax import lax
from jax.experimental import pallas as pl
from jax.experimental.pallas import tpu as pltpu
from jax.experimental.pallas import tpu_sc as plsc

_LANES = 128


def _make_gather(num_rows: int, num_ids_rows: int, num_ids_cols: int, dim: int):
    info = plsc.get_sparse_core_info()
    nc, ns = info.num_cores, info.num_subcores
    nw = nc * ns
    batch = num_ids_rows * num_ids_cols
    b_per_w = batch // nw
    per_row = num_ids_cols // b_per_w
    mesh = plsc.VectorSubcoreMesh(core_axis_name="c", subcore_axis_name="s")

    chunk = 128
    n_chunks = b_per_w // chunk

    @functools.partial(
        pl.kernel,
        mesh=mesh,
        out_type=jax.ShapeDtypeStruct((num_ids_rows, num_ids_cols, dim), jnp.float32),
        scratch_types=[
            pltpu.VMEM((b_per_w,), jnp.int32),
            pltpu.VMEM((2, chunk, _LANES), jnp.float32),
            pltpu.VMEM((2, chunk, dim), jnp.float32),
            [pltpu.SemaphoreType.DMA] * 2,
            [pltpu.SemaphoreType.DMA] * 2,
        ],
    )
    def gather_kernel(table_hbm, idx_hbm, out_hbm, idx_v, rows_w, rows_c, gsem, ssem):
        wid = lax.axis_index("s") * nc + lax.axis_index("c")
        r = wid // per_row
        col = (wid % per_row) * b_per_w
        pltpu.sync_copy(idx_hbm.at[r, pl.ds(col, b_per_w)], idx_v)

        def start_gather(c):
            pltpu.async_copy(
                table_hbm.at[idx_v.at[pl.ds(c * chunk, chunk)]],
                rows_w.at[c % 2],
                gsem[c % 2],
            )

        def wait_gather(c):
            pltpu.make_async_copy(
                table_hbm.at[idx_v.at[pl.ds(c * chunk, chunk)]],
                rows_w.at[c % 2],
                gsem[c % 2],
            ).wait()

        def start_store(c):
            pltpu.async_copy(
                rows_c.at[c % 2],
                out_hbm.at[r, pl.ds(col + c * chunk, chunk)],
                ssem[c % 2],
            )

        def wait_store(c):
            pltpu.make_async_copy(
                rows_c.at[c % 2],
                out_hbm.at[r, pl.ds(col + c * chunk, chunk)],
                ssem[c % 2],
            ).wait()

        start_gather(0)
        for c in range(n_chunks):
            b = c % 2
            if c + 1 < n_chunks:
                start_gather(c + 1)
            wait_gather(c)
            if c >= 2:
                wait_store(c - 2)

            def compact_row(i, carry):
                for j in range(dim // 16):
                    rows_c[b, i, pl.ds(j * 16, 16)] = rows_w[b, i, pl.ds(j * 16, 16)]
                return carry

            lax.fori_loop(0, chunk, compact_row, 0)
            start_store(c)
        for c in range(n_chunks - 2, n_chunks):
            wait_store(c)

    return gather_kernel


def kernel(embed_id, weight):
    num_rows, dim = weight.shape
    ir, ic = embed_id.shape
    wpad = jnp.pad(weight, ((0, 0), (0, _LANES - dim)))
    out = _make_gather(num_rows, ir, ic, dim)(wpad, embed_id.astype(jnp.int32))
    return out
